# baseline (speedup 1.0000x reference)
import functools

import jax
import jax.numpy as jnp
from jax import lax
from jax.experimental import pallas as pl
from jax.experimental.pallas import tpu as pltpu
from jax.experimental.pallas import tpu_sc as plsc

D_MODEL = 64
SCALE = 8.0

NUM_CORES = 2
NUM_SUBCORES = 16
NUM_WORKERS = NUM_CORES * NUM_SUBCORES

B_TOKENS = 4096
P_POS = 200
STRIPE = B_TOKENS // NUM_WORKERS
IDX_PER_WORKER = STRIPE * P_POS
OUT_ROWS = P_POS * 8 * NUM_WORKERS * 8
LANES = 16
NBUF = 2


def _emb_body(x_hbm, lut_hbm, out_hbm, idx_v, gl0, gl1, rows0, rows1,
              st0, st1, gs0, gs1, os0, os1):
    wid = lax.axis_index("s") * NUM_CORES + lax.axis_index("c")
    glist = (gl0, gl1)
    rows = (rows0, rows1)
    stage = (st0, st1)
    gsem = (gs0, gs1)
    osem = (os0, os1)

    pltpu.sync_copy(x_hbm.at[pl.ds(wid * IDX_PER_WORKER, IDX_PER_WORKER)],
                    idx_v)

    iota = lax.iota(jnp.int32, LANES)
    iota_p = iota * P_POS
    rvecs = [iota + j * LANES for j in range(D_MODEL // LANES)]

    def build_glist(p, b):
        for s in range(8):
            ivec = iota_p + (s * (LANES * P_POS) + p)
            g16 = plsc.load_gather(idx_v, [ivec])
            glist[b][pl.ds(s * LANES, LANES)] = g16 * 2

    def start_gather(b):
        pltpu.async_copy(lut_hbm.at[glist[b]], rows[b], gsem[b])

    def wait_gather(b):
        pltpu.make_async_copy(lut_hbm.at[glist[b]], rows[b], gsem[b]).wait()

    def transpose_scale(b):
        src = rows[b]
        dst = stage[b]

        @plsc.parallel_loop(0, STRIPE, step=1, unroll=4)
        def _(t):
            cvec = jnp.full((LANES,), t, jnp.int32)
            for j in range(D_MODEL // LANES):
                v = src[t, pl.ds(j * LANES, LANES)] * SCALE
                plsc.store_scatter(dst, [rvecs[j], cvec], v)

    def start_out(p, b):
        base = p * 2048 + wid * 8
        for td in range(8):
            pltpu.async_copy(stage[b].at[pl.ds(td * 8, 8)],
                             out_hbm.at[pl.ds(base + td * 256, 8)], osem[b])

    def wait_out(p, b):
        base = p * 2048 + wid * 8
        for td in range(8):
            pltpu.make_async_copy(stage[b].at[pl.ds(td * 8, 8)],
                                  out_hbm.at[pl.ds(base + td * 256, 8)],
                                  osem[b]).wait()

    build_glist(0, 0)
    start_gather(0)
    build_glist(1, 1)
    start_gather(1)

    @pl.loop(0, P_POS, step=NBUF)
    def _(p):
        for b in range(NBUF):
            pp = p + b
            wait_gather(b)

            @pl.when(pp >= NBUF)
            def _():
                wait_out(pp - NBUF, b)

            transpose_scale(b)
            start_out(pp, b)

            @pl.when(pp + NBUF < P_POS)
            def _():
                build_glist(pp + NBUF, b)
                start_gather(b)

    for b in range(NBUF):
        wait_out(P_POS - NBUF + b, b)


_emb = functools.partial(
    pl.kernel,
    out_type=jax.ShapeDtypeStruct((OUT_ROWS, 128), jnp.float32),
    mesh=plsc.VectorSubcoreMesh(
        core_axis_name="c",
        subcore_axis_name="s",
        num_cores=NUM_CORES,
        num_subcores=NUM_SUBCORES,
    ),
    scratch_types=[
        pltpu.VMEM((IDX_PER_WORKER,), jnp.int32),
        pltpu.VMEM((STRIPE,), jnp.int32),
        pltpu.VMEM((STRIPE,), jnp.int32),
        pltpu.VMEM((STRIPE, D_MODEL), jnp.float32),
        pltpu.VMEM((STRIPE, D_MODEL), jnp.float32),
        pltpu.VMEM((D_MODEL, STRIPE), jnp.float32),
        pltpu.VMEM((D_MODEL, STRIPE), jnp.float32),
        pltpu.SemaphoreType.DMA,
        pltpu.SemaphoreType.DMA,
        pltpu.SemaphoreType.DMA,
        pltpu.SemaphoreType.DMA,
    ],
    compiler_params=pltpu.CompilerParams(use_tc_tiling_on_sc=False, needs_layout_passes=False),
)(_emb_body)


@jax.jit
def kernel(x, lut):
    lut_padded = jnp.pad(lut, ((0, 0), (0, 128 - D_MODEL)))
    lut_lin = lut_padded.reshape(2 * 1000000, D_MODEL)
    flat = _emb(x.reshape(-1), lut_lin)
    o5 = flat.reshape(P_POS, 8, NUM_WORKERS, 8, 128)
    return jnp.transpose(o5, (2, 4, 0, 1, 3)).reshape(
        B_TOKENS, P_POS, D_MODEL)


# speedup vs baseline: 1.4569x; 1.4569x over previous
import functools

import jax
import jax.numpy as jnp
from jax import lax
from jax.experimental import pallas as pl
from jax.experimental.pallas import tpu as pltpu
from jax.experimental.pallas import tpu_sc as plsc

D_MODEL = 64
D_PAD = 128
SCALE = 8.0

NUM_CORES = 2
NUM_SUBCORES = 16
NUM_WORKERS = NUM_CORES * NUM_SUBCORES

B_TOTAL = 4096 * 200
ROWS_PER_WORKER = B_TOTAL // NUM_WORKERS
CHUNK = 400
NUM_CHUNKS = ROWS_PER_WORKER // CHUNK
NBUF = 2
LANES = 16


def _scale_chunk(src, dst):
    @plsc.parallel_loop(0, CHUNK, step=1, unroll=8)
    def _(r):
        for j in range(D_MODEL // LANES):
            sl = pl.ds(j * LANES, LANES)
            dst[r, sl] = src[r, sl] * SCALE


def _emb_body(x_hbm, lut_hbm, out_hbm, idx_v,
              rows0, rows1, stage0, stage1, gs0, gs1, os0, os1):
    wid = lax.axis_index("s") * NUM_CORES + lax.axis_index("c")
    base = wid * ROWS_PER_WORKER
    rows = (rows0, rows1)
    stage = (stage0, stage1)
    gsem = (gs0, gs1)
    osem = (os0, os1)

    pltpu.sync_copy(x_hbm.at[pl.ds(base, ROWS_PER_WORKER)], idx_v)

    @plsc.parallel_loop(0, ROWS_PER_WORKER // LANES, step=1, unroll=8)
    def _(i):
        sl = pl.ds(i * LANES, LANES)
        idx_v[sl] = idx_v[sl] * 2

    def start_gather(g, b):
        pltpu.async_copy(
            lut_hbm.at[idx_v.at[pl.ds(g * CHUNK, CHUNK)]], rows[b], gsem[b])

    def wait_gather(g, b):
        pltpu.make_async_copy(
            lut_hbm.at[idx_v.at[pl.ds(g * CHUNK, CHUNK)]], rows[b],
            gsem[b]).wait()

    def start_out(g, b):
        pltpu.async_copy(
            stage[b],
            out_hbm.at[pl.ds(base + g * CHUNK, CHUNK), pl.ds(0, D_MODEL)],
            osem[b])

    def wait_out(g, b):
        pltpu.make_async_copy(
            stage[b],
            out_hbm.at[pl.ds(base + g * CHUNK, CHUNK), pl.ds(0, D_MODEL)],
            osem[b]).wait()

    start_gather(0, 0)
    start_gather(1, 1)

    @pl.loop(0, NUM_CHUNKS, step=NBUF)
    def _(g):
        for b in range(NBUF):
            gg = g + b
            wait_gather(gg, b)

            @pl.when(gg >= NBUF)
            def _():
                wait_out(gg - NBUF, b)

            _scale_chunk(rows[b], stage[b])

            @pl.when(gg + NBUF < NUM_CHUNKS)
            def _():
                start_gather(gg + NBUF, b)

            start_out(gg, b)

    for b in range(NBUF):
        wait_out(NUM_CHUNKS - NBUF + b, b)


_emb = functools.partial(
    pl.kernel,
    out_type=jax.ShapeDtypeStruct((B_TOTAL, D_PAD), jnp.float32),
    mesh=plsc.VectorSubcoreMesh(
        core_axis_name="c",
        subcore_axis_name="s",
        num_cores=NUM_CORES,
        num_subcores=NUM_SUBCORES,
    ),
    scratch_types=[
        pltpu.VMEM((ROWS_PER_WORKER,), jnp.int32),
        pltpu.VMEM((CHUNK, D_MODEL), jnp.float32),
        pltpu.VMEM((CHUNK, D_MODEL), jnp.float32),
        pltpu.VMEM((CHUNK, D_MODEL), jnp.float32),
        pltpu.VMEM((CHUNK, D_MODEL), jnp.float32),
        pltpu.SemaphoreType.DMA,
        pltpu.SemaphoreType.DMA,
        pltpu.SemaphoreType.DMA,
        pltpu.SemaphoreType.DMA,
    ],
    compiler_params=pltpu.CompilerParams(use_tc_tiling_on_sc=False,
                                         needs_layout_passes=False),
)(_emb_body)


@jax.jit
def kernel(x, lut):
    lut_padded = jnp.pad(lut, ((0, 0), (0, D_PAD - D_MODEL)))
    lut_lin = lut_padded.reshape(2 * 1000000, D_MODEL)
    flat = _emb(x.reshape(-1), lut_lin)
    return flat[:, :D_MODEL].reshape(x.shape + (D_MODEL,))
